# baseline (speedup 1.0000x reference)
import functools

import jax
import jax.numpy as jnp
from jax import lax
from jax.experimental import pallas as pl
from jax.experimental.pallas import tpu as pltpu

BUCKET_N = 64
HALF_N = 32


def _bucket_body(q_ref, r_ref, out_ref):
    q = q_ref[0]
    r = r_ref[0]
    xr = jnp.dot(q, r, preferred_element_type=jnp.float32)
    vals = jnp.concatenate([xr, -xr], axis=1)
    m = jnp.max(vals, axis=1, keepdims=True)
    cols = lax.broadcasted_iota(jnp.int32, vals.shape, 1)
    b = jnp.min(jnp.where(vals == m, cols, BUCKET_N), axis=1)
    out_ref[0, 0, :] = b


def _compute_buckets(query, R, interpret=False):
    B, S, d = query.shape
    return pl.pallas_call(
        _bucket_body,
        grid=(B,),
        in_specs=[
            pl.BlockSpec((1, S, d), lambda i: (i, 0, 0)),
            pl.BlockSpec((1, d, HALF_N), lambda i: (i, 0, 0)),
        ],
        out_specs=pl.BlockSpec((1, 1, S), lambda i: (i, 0, 0)),
        out_shape=jax.ShapeDtypeStruct((B, 1, S), jnp.int32),
        interpret=interpret,
    )(query, R)


def kernel(query, key, value):
    B, S, d = query.shape
    rkey = jax.random.key(42)
    R = jax.random.normal(rkey, (B, d, BUCKET_N // 2), dtype=query.dtype)
    buckets = _compute_buckets(query, R)[:, 0, :]
    sticker = jnp.argsort(buckets, axis=-1)
    return sticker
